# baseline (speedup 1.0000x reference)
import jax
import jax.numpy as jnp
from jax import lax
from jax.experimental import pallas as pl
from jax.experimental.pallas import tpu as pltpu
from jax.experimental.pallas import tpu_sc as plsc

N_NODES = 10000
N_EDGES = 320000
D = 128

NC = 2
NS = 16
NW = NC * NS

E_TILE_DEG = N_EDGES // NS
DEG_CHUNK = 2000
N_HIST = NS * 640

E_PAD = 327680
E_C0 = 81920
T_C0 = E_C0 // NS
T_C1 = (E_PAD - E_C0) // NS
AGG_CHUNK = 64
N_ACC = NS * 632
ROWS_T = N_ACC // NS


def _deg_body(edges, out, ibuf0, ibuf1, hist, acc, tmp, shist, isem):
    ibufs = (ibuf0, ibuf1)
    c = lax.axis_index("c")
    s = lax.axis_index("s")
    zeros16 = jnp.zeros((16,), jnp.float32)
    ones16 = jnp.ones((16,), jnp.float32)

    def _zero(i, carry):
        hist[pl.ds(i * 16, 16)] = zeros16
        return carry

    lax.fori_loop(0, N_HIST // 16, _zero, 0)

    base = c * N_EDGES + s * E_TILE_DEG
    n_chunks = E_TILE_DEG // DEG_CHUNK

    def _start_idx(k, b):
        off = pl.multiple_of(base + k * DEG_CHUNK, 8)
        pltpu.async_copy(edges.at[pl.ds(off, DEG_CHUNK)], ibufs[b], isem.at[b])

    def _wait_idx(b):
        pltpu.make_async_copy(
            edges.at[pl.ds(0, DEG_CHUNK)], ibufs[b], isem.at[b]
        ).wait()

    _start_idx(0, 0)
    _start_idx(1, 1)

    def _chunk(g, carry):
        for b in (0, 1):
            k = 2 * g + b
            _wait_idx(b)

            ib = ibufs[b]

            def _scan(j, c2):
                idx = ib[pl.ds(j * 16, 16)]
                plsc.addupdate_scatter(hist, [idx], ones16)
                return c2

            lax.fori_loop(0, DEG_CHUNK // 16, _scan, 0)

            @pl.when(k + 2 < n_chunks)
            def _():
                _start_idx(k + 2, b)

        return carry

    lax.fori_loop(0, n_chunks // 2, _chunk, 0)

    pltpu.sync_copy(hist, shist.at[s])
    plsc.subcore_barrier()

    rb = pl.multiple_of(s * 640, 8)
    pltpu.sync_copy(shist.at[0, pl.ds(rb, 640)], acc)

    def _reduce(h, carry):
        pltpu.sync_copy(shist.at[h, pl.ds(rb, 640)], tmp)

        def _addv(i, c2):
            o = i * 16
            acc[pl.ds(o, 16)] = acc[pl.ds(o, 16)] + tmp[pl.ds(o, 16)]
            return c2

        lax.fori_loop(0, 640 // 16, _addv, 0)
        return carry

    lax.fori_loop(1, NS, _reduce, 0)
    ob = pl.multiple_of(c * N_HIST + s * 640, 8)
    pltpu.sync_copy(acc, out.at[pl.ds(ob, 640)])


_deg_call = pl.kernel(
    _deg_body,
    out_type=jax.ShapeDtypeStruct((2 * N_HIST,), jnp.float32),
    mesh=plsc.VectorSubcoreMesh(core_axis_name="c", subcore_axis_name="s"),
    scratch_types=[
        pltpu.VMEM((DEG_CHUNK,), jnp.int32),
        pltpu.VMEM((DEG_CHUNK,), jnp.int32),
        pltpu.VMEM((N_HIST,), jnp.float32),
        pltpu.VMEM((640,), jnp.float32),
        pltpu.VMEM((640,), jnp.float32),
        pltpu.VMEM_SHARED((NS, N_HIST), jnp.float32),
        pltpu.SemaphoreType.DMA((2,)),
    ],
    compiler_params=pltpu.CompilerParams(needs_layout_passes=False),
)


NBUF = 4


def _agg_body(h, srcp, dstp, zrows, out, sidx, didx, msg, acc_sh,
              sisem, disem, gsem, ssem):
    c = lax.axis_index("c")
    s = lax.axis_index("s")
    zb = pl.multiple_of(s * ROWS_T, 8)
    pltpu.sync_copy(zrows.at[pl.ds(zb, ROWS_T)], acc_sh.at[pl.ds(zb, ROWS_T)])
    plsc.subcore_barrier()

    t_len = T_C0 + c * (T_C1 - T_C0)
    eb = c * E_C0 + s * t_len
    n_chunks = t_len // AGG_CHUNK

    def _off(k):
        return pl.multiple_of(eb + k * AGG_CHUNK, 8)

    def _start_sidx(k, b):
        pltpu.async_copy(srcp.at[pl.ds(_off(k), AGG_CHUNK)], sidx.at[b], sisem.at[b])

    def _wait_sidx(b):
        pltpu.make_async_copy(srcp.at[pl.ds(0, AGG_CHUNK)], sidx.at[b], sisem.at[b]).wait()

    def _start_didx(k, b):
        pltpu.async_copy(dstp.at[pl.ds(_off(k), AGG_CHUNK)], didx.at[b], disem.at[b])

    def _wait_didx(b):
        pltpu.make_async_copy(dstp.at[pl.ds(0, AGG_CHUNK)], didx.at[b], disem.at[b]).wait()

    def _start_gather(b):
        pltpu.async_copy(h.at[sidx.at[b]], msg.at[b], gsem.at[b])

    def _wait_gather(b):
        pltpu.make_async_copy(h.at[sidx.at[b]], msg.at[b], gsem.at[b]).wait()

    def _start_scatter(b):
        pltpu.async_copy(msg.at[b], acc_sh.at[didx.at[b]], ssem.at[b], add=True)

    def _wait_scatter(b):
        pltpu.make_async_copy(msg.at[b], acc_sh.at[didx.at[b]], ssem.at[b]).wait()

    for b in range(NBUF):
        _start_sidx(b, b)
    _start_didx(0, 0)
    _start_didx(1, 1)
    _wait_sidx(0)
    _start_gather(0)
    _wait_sidx(1)
    _start_gather(1)

    def _iter(g, carry):
        for b in range(NBUF):
            k = NBUF * g + b
            b2 = (b + 2) % NBUF

            _wait_gather(b)
            _wait_didx(b)
            _start_scatter(b)

            @pl.when(k + 4 < n_chunks)
            def _():
                _start_sidx(k + 4, b)

            @pl.when(k >= 2)
            def _():
                _wait_scatter(b2)

            @pl.when(k + 2 < n_chunks)
            def _():
                _wait_sidx(b2)
                _start_gather(b2)
                _start_didx(k + 2, b2)

        return carry

    lax.fori_loop(0, n_chunks // NBUF, _iter, 0)
    _wait_scatter(NBUF - 2)
    _wait_scatter(NBUF - 1)
    plsc.subcore_barrier()

    ob = pl.multiple_of(s * ROWS_T, 8)
    pltpu.sync_copy(acc_sh.at[pl.ds(ob, ROWS_T)], out.at[c, pl.ds(ob, ROWS_T)])


_agg_call = pl.kernel(
    _agg_body,
    out_type=jax.ShapeDtypeStruct((2, N_ACC, D), jnp.float32),
    mesh=plsc.VectorSubcoreMesh(core_axis_name="c", subcore_axis_name="s"),
    scratch_types=[
        pltpu.VMEM((NBUF, AGG_CHUNK), jnp.int32),
        pltpu.VMEM((NBUF, AGG_CHUNK), jnp.int32),
        pltpu.VMEM((NBUF, AGG_CHUNK, D), jnp.float32),
        pltpu.VMEM_SHARED((N_ACC, D), jnp.float32),
        pltpu.SemaphoreType.DMA((NBUF,)),
        pltpu.SemaphoreType.DMA((NBUF,)),
        pltpu.SemaphoreType.DMA((NBUF,)),
        pltpu.SemaphoreType.DMA((NBUF,)),
    ],
    compiler_params=pltpu.CompilerParams(needs_layout_passes=False),
)


def _scale_body(x_ref, deg_ref, o_ref):
    norm = lax.rsqrt(jnp.maximum(deg_ref[...], 1.0))
    o_ref[...] = x_ref[...] * norm


def _mid_body(parts_ref, dd_ref, ds_ref, w_ref, o_ref):
    agg = parts_ref[0, :N_NODES, :] + parts_ref[1, :N_NODES, :]
    nd = lax.rsqrt(jnp.maximum(dd_ref[...], 1.0))
    ns = lax.rsqrt(jnp.maximum(ds_ref[...], 1.0))
    h = jnp.dot(agg * nd, w_ref[...], preferred_element_type=jnp.float32)
    o_ref[...] = jnp.maximum(h, 0.0) * ns


def _out_body(parts_ref, dd_ref, w_ref, o_ref):
    agg = parts_ref[0, :N_NODES, :] + parts_ref[1, :N_NODES, :]
    nd = lax.rsqrt(jnp.maximum(dd_ref[...], 1.0))
    o_ref[...] = jnp.dot(agg * nd, w_ref[...], preferred_element_type=jnp.float32)


_scale_call = pl.pallas_call(
    _scale_body,
    out_shape=jax.ShapeDtypeStruct((N_NODES, D), jnp.float32),
)

_mid_call = pl.pallas_call(
    _mid_body,
    out_shape=jax.ShapeDtypeStruct((N_NODES, D), jnp.float32),
)

_out_call = pl.pallas_call(
    _out_body,
    out_shape=jax.ShapeDtypeStruct((N_NODES, D), jnp.float32),
)


def kernel(x, edge_index, W1, W2):
    ei = edge_index.astype(jnp.int32)
    n_pad = E_PAD - N_EDGES
    srcp = jnp.concatenate([ei[0], jnp.zeros((n_pad,), jnp.int32)])
    dstp = jnp.concatenate([ei[1], jnp.full((n_pad,), N_NODES, jnp.int32)])
    zrows = jnp.zeros((N_ACC, D), jnp.float32)

    edges_flat = jnp.concatenate([ei[0], ei[1]])
    degs = _deg_call(edges_flat)
    deg_src = degs[:N_NODES].reshape(N_NODES, 1)
    deg_dst = degs[N_HIST:N_HIST + N_NODES].reshape(N_NODES, 1)

    h0 = _scale_call(x, deg_src)
    parts1 = _agg_call(h0, srcp, dstp, zrows)
    h1 = _mid_call(parts1, deg_dst, deg_src, W1)
    parts2 = _agg_call(h1, srcp, dstp, zrows)
    return _out_call(parts2, deg_dst, W2)


# speedup vs baseline: 1.0867x; 1.0867x over previous
import jax
import jax.numpy as jnp
from jax import lax
from jax.experimental import pallas as pl
from jax.experimental.pallas import tpu as pltpu
from jax.experimental.pallas import tpu_sc as plsc

N_NODES = 10000
N_EDGES = 320000
D = 128

NC = 2
NS = 16
NW = NC * NS

E_TILE_DEG = N_EDGES // NS
DEG_CHUNK = 2000
N_HIST = NS * 640

E_PAD = 327680
E_C0 = 245760
T_C0 = E_C0 // NS
T_C1 = (E_PAD - E_C0) // NS
AGG_CHUNK = 64
N_ACC = NS * 632
ROWS_T = N_ACC // NS


def _deg_body(edges, out, ibuf0, ibuf1, hist, acc, tmp, shist, isem):
    ibufs = (ibuf0, ibuf1)
    c = lax.axis_index("c")
    s = lax.axis_index("s")
    zeros16 = jnp.zeros((16,), jnp.float32)
    ones16 = jnp.ones((16,), jnp.float32)

    def _zero(i, carry):
        hist[pl.ds(i * 16, 16)] = zeros16
        return carry

    lax.fori_loop(0, N_HIST // 16, _zero, 0)

    base = c * N_EDGES + s * E_TILE_DEG
    n_chunks = E_TILE_DEG // DEG_CHUNK

    def _start_idx(k, b):
        off = pl.multiple_of(base + k * DEG_CHUNK, 8)
        pltpu.async_copy(edges.at[pl.ds(off, DEG_CHUNK)], ibufs[b], isem.at[b])

    def _wait_idx(b):
        pltpu.make_async_copy(
            edges.at[pl.ds(0, DEG_CHUNK)], ibufs[b], isem.at[b]
        ).wait()

    _start_idx(0, 0)
    _start_idx(1, 1)

    def _chunk(g, carry):
        for b in (0, 1):
            k = 2 * g + b
            _wait_idx(b)

            ib = ibufs[b]

            def _scan(j, c2):
                idx = ib[pl.ds(j * 16, 16)]
                plsc.addupdate_scatter(hist, [idx], ones16)
                return c2

            lax.fori_loop(0, DEG_CHUNK // 16, _scan, 0)

            @pl.when(k + 2 < n_chunks)
            def _():
                _start_idx(k + 2, b)

        return carry

    lax.fori_loop(0, n_chunks // 2, _chunk, 0)

    pltpu.sync_copy(hist, shist.at[s])
    plsc.subcore_barrier()

    rb = pl.multiple_of(s * 640, 8)
    pltpu.sync_copy(shist.at[0, pl.ds(rb, 640)], acc)

    def _reduce(h, carry):
        pltpu.sync_copy(shist.at[h, pl.ds(rb, 640)], tmp)

        def _addv(i, c2):
            o = i * 16
            acc[pl.ds(o, 16)] = acc[pl.ds(o, 16)] + tmp[pl.ds(o, 16)]
            return c2

        lax.fori_loop(0, 640 // 16, _addv, 0)
        return carry

    lax.fori_loop(1, NS, _reduce, 0)
    ob = pl.multiple_of(c * N_HIST + s * 640, 8)
    pltpu.sync_copy(acc, out.at[pl.ds(ob, 640)])


_deg_call = pl.kernel(
    _deg_body,
    out_type=jax.ShapeDtypeStruct((2 * N_HIST,), jnp.float32),
    mesh=plsc.VectorSubcoreMesh(core_axis_name="c", subcore_axis_name="s"),
    scratch_types=[
        pltpu.VMEM((DEG_CHUNK,), jnp.int32),
        pltpu.VMEM((DEG_CHUNK,), jnp.int32),
        pltpu.VMEM((N_HIST,), jnp.float32),
        pltpu.VMEM((640,), jnp.float32),
        pltpu.VMEM((640,), jnp.float32),
        pltpu.VMEM_SHARED((NS, N_HIST), jnp.float32),
        pltpu.SemaphoreType.DMA((2,)),
    ],
    compiler_params=pltpu.CompilerParams(needs_layout_passes=False),
)


NBUF = 4


def _agg_body(h, srcp, dstp, zrows, out, sidx, didx, msg, acc_sh,
              sisem, disem, gsem, ssem):
    c = lax.axis_index("c")
    s = lax.axis_index("s")
    zb = pl.multiple_of(s * ROWS_T, 8)
    pltpu.sync_copy(zrows.at[pl.ds(zb, ROWS_T)], acc_sh.at[pl.ds(zb, ROWS_T)])
    plsc.subcore_barrier()

    t_len = T_C0 + c * (T_C1 - T_C0)
    eb = c * E_C0 + s * t_len
    n_chunks = t_len // AGG_CHUNK

    def _off(k):
        return pl.multiple_of(eb + k * AGG_CHUNK, 8)

    def _start_sidx(k, b):
        pltpu.async_copy(srcp.at[pl.ds(_off(k), AGG_CHUNK)], sidx.at[b], sisem.at[b])

    def _wait_sidx(b):
        pltpu.make_async_copy(srcp.at[pl.ds(0, AGG_CHUNK)], sidx.at[b], sisem.at[b]).wait()

    def _start_didx(k, b):
        pltpu.async_copy(dstp.at[pl.ds(_off(k), AGG_CHUNK)], didx.at[b], disem.at[b])

    def _wait_didx(b):
        pltpu.make_async_copy(dstp.at[pl.ds(0, AGG_CHUNK)], didx.at[b], disem.at[b]).wait()

    def _start_gather(b):
        pltpu.async_copy(h.at[sidx.at[b]], msg.at[b], gsem.at[b])

    def _wait_gather(b):
        pltpu.make_async_copy(h.at[sidx.at[b]], msg.at[b], gsem.at[b]).wait()

    def _start_scatter(b):
        pltpu.async_copy(msg.at[b], acc_sh.at[didx.at[b]], ssem.at[b], add=True)

    def _wait_scatter(b):
        pltpu.make_async_copy(msg.at[b], acc_sh.at[didx.at[b]], ssem.at[b]).wait()

    for b in range(NBUF):
        _start_sidx(b, b)
    _start_didx(0, 0)
    _start_didx(1, 1)
    _wait_sidx(0)
    _start_gather(0)
    _wait_sidx(1)
    _start_gather(1)

    def _iter(g, carry):
        for b in range(NBUF):
            k = NBUF * g + b
            b2 = (b + 2) % NBUF

            _wait_gather(b)
            _wait_didx(b)
            _start_scatter(b)

            @pl.when(k + 4 < n_chunks)
            def _():
                _start_sidx(k + 4, b)

            @pl.when(k >= 2)
            def _():
                _wait_scatter(b2)

            @pl.when(k + 2 < n_chunks)
            def _():
                _wait_sidx(b2)
                _start_gather(b2)
                _start_didx(k + 2, b2)

        return carry

    lax.fori_loop(0, n_chunks // NBUF, _iter, 0)
    _wait_scatter(NBUF - 2)
    _wait_scatter(NBUF - 1)
    plsc.subcore_barrier()

    ob = pl.multiple_of(s * ROWS_T, 8)
    pltpu.sync_copy(acc_sh.at[pl.ds(ob, ROWS_T)], out.at[c, pl.ds(ob, ROWS_T)])


_agg_call = pl.kernel(
    _agg_body,
    out_type=jax.ShapeDtypeStruct((2, N_ACC, D), jnp.float32),
    mesh=plsc.VectorSubcoreMesh(core_axis_name="c", subcore_axis_name="s"),
    scratch_types=[
        pltpu.VMEM((NBUF, AGG_CHUNK), jnp.int32),
        pltpu.VMEM((NBUF, AGG_CHUNK), jnp.int32),
        pltpu.VMEM((NBUF, AGG_CHUNK, D), jnp.float32),
        pltpu.VMEM_SHARED((N_ACC, D), jnp.float32),
        pltpu.SemaphoreType.DMA((NBUF,)),
        pltpu.SemaphoreType.DMA((NBUF,)),
        pltpu.SemaphoreType.DMA((NBUF,)),
        pltpu.SemaphoreType.DMA((NBUF,)),
    ],
    compiler_params=pltpu.CompilerParams(needs_layout_passes=False),
)


def _scale_body(x_ref, deg_ref, o_ref):
    norm = lax.rsqrt(jnp.maximum(deg_ref[...], 1.0))
    o_ref[...] = x_ref[...] * norm


def _mid_body(parts_ref, dd_ref, ds_ref, w_ref, o_ref):
    agg = parts_ref[0, :N_NODES, :] + parts_ref[1, :N_NODES, :]
    nd = lax.rsqrt(jnp.maximum(dd_ref[...], 1.0))
    ns = lax.rsqrt(jnp.maximum(ds_ref[...], 1.0))
    h = jnp.dot(agg * nd, w_ref[...], preferred_element_type=jnp.float32)
    o_ref[...] = jnp.maximum(h, 0.0) * ns


def _out_body(parts_ref, dd_ref, w_ref, o_ref):
    agg = parts_ref[0, :N_NODES, :] + parts_ref[1, :N_NODES, :]
    nd = lax.rsqrt(jnp.maximum(dd_ref[...], 1.0))
    o_ref[...] = jnp.dot(agg * nd, w_ref[...], preferred_element_type=jnp.float32)


_scale_call = pl.pallas_call(
    _scale_body,
    out_shape=jax.ShapeDtypeStruct((N_NODES, D), jnp.float32),
)

_mid_call = pl.pallas_call(
    _mid_body,
    out_shape=jax.ShapeDtypeStruct((N_NODES, D), jnp.float32),
)

_out_call = pl.pallas_call(
    _out_body,
    out_shape=jax.ShapeDtypeStruct((N_NODES, D), jnp.float32),
)


def kernel(x, edge_index, W1, W2):
    ei = edge_index.astype(jnp.int32)
    n_pad = E_PAD - N_EDGES
    srcp = jnp.concatenate([ei[0], jnp.zeros((n_pad,), jnp.int32)])
    dstp = jnp.concatenate([ei[1], jnp.full((n_pad,), N_NODES, jnp.int32)])
    zrows = jnp.zeros((N_ACC, D), jnp.float32)

    edges_flat = jnp.concatenate([ei[0], ei[1]])
    degs = _deg_call(edges_flat)
    deg_src = degs[:N_NODES].reshape(N_NODES, 1)
    deg_dst = degs[N_HIST:N_HIST + N_NODES].reshape(N_NODES, 1)

    h0 = _scale_call(x, deg_src)
    parts1 = _agg_call(h0, srcp, dstp, zrows)
    h1 = _mid_call(parts1, deg_dst, deg_src, W1)
    parts2 = _agg_call(h1, srcp, dstp, zrows)
    return _out_call(parts2, deg_dst, W2)


# speedup vs baseline: 1.0960x; 1.0085x over previous
import jax
import jax.numpy as jnp
from jax import lax
from jax.experimental import pallas as pl
from jax.experimental.pallas import tpu as pltpu
from jax.experimental.pallas import tpu_sc as plsc

N_NODES = 10000
N_EDGES = 320000
D = 128

NC = 2
NS = 16
NW = NC * NS

E_TILE_DEG = N_EDGES // NS
DEG_CHUNK = 2000
N_HIST = NS * 640

E_PAD = 327680
E_C0 = 245760
T_C0 = E_C0 // NS
T_C1 = (E_PAD - E_C0) // NS
AGG_CHUNK = 64
N_ACC = NS * 632
ROWS_T = N_ACC // NS


def _deg_body(edges, out, ibuf0, ibuf1, hist, acc, tmp, shist, isem):
    ibufs = (ibuf0, ibuf1)
    c = lax.axis_index("c")
    s = lax.axis_index("s")
    zeros16 = jnp.zeros((16,), jnp.float32)
    ones16 = jnp.ones((16,), jnp.float32)

    def _zero(i, carry):
        hist[pl.ds(i * 16, 16)] = zeros16
        return carry

    lax.fori_loop(0, N_HIST // 16, _zero, 0)

    base = c * N_EDGES + s * E_TILE_DEG
    n_chunks = E_TILE_DEG // DEG_CHUNK

    def _start_idx(k, b):
        off = pl.multiple_of(base + k * DEG_CHUNK, 8)
        pltpu.async_copy(edges.at[pl.ds(off, DEG_CHUNK)], ibufs[b], isem.at[b])

    def _wait_idx(b):
        pltpu.make_async_copy(
            edges.at[pl.ds(0, DEG_CHUNK)], ibufs[b], isem.at[b]
        ).wait()

    _start_idx(0, 0)
    _start_idx(1, 1)

    def _chunk(g, carry):
        for b in (0, 1):
            k = 2 * g + b
            _wait_idx(b)

            ib = ibufs[b]

            def _scan(j, c2):
                idx = ib[pl.ds(j * 16, 16)]
                plsc.addupdate_scatter(hist, [idx], ones16)
                return c2

            lax.fori_loop(0, DEG_CHUNK // 16, _scan, 0)

            @pl.when(k + 2 < n_chunks)
            def _():
                _start_idx(k + 2, b)

        return carry

    lax.fori_loop(0, n_chunks // 2, _chunk, 0)

    pltpu.sync_copy(hist, shist.at[s])
    plsc.subcore_barrier()

    rb = pl.multiple_of(s * 640, 8)
    pltpu.sync_copy(shist.at[0, pl.ds(rb, 640)], acc)

    def _reduce(h, carry):
        pltpu.sync_copy(shist.at[h, pl.ds(rb, 640)], tmp)

        def _addv(i, c2):
            o = i * 16
            acc[pl.ds(o, 16)] = acc[pl.ds(o, 16)] + tmp[pl.ds(o, 16)]
            return c2

        lax.fori_loop(0, 640 // 16, _addv, 0)
        return carry

    lax.fori_loop(1, NS, _reduce, 0)
    ob = pl.multiple_of(c * N_HIST + s * 640, 8)
    pltpu.sync_copy(acc, out.at[pl.ds(ob, 640)])


_deg_call = pl.kernel(
    _deg_body,
    out_type=jax.ShapeDtypeStruct((2 * N_HIST,), jnp.float32),
    mesh=plsc.VectorSubcoreMesh(core_axis_name="c", subcore_axis_name="s"),
    scratch_types=[
        pltpu.VMEM((DEG_CHUNK,), jnp.int32),
        pltpu.VMEM((DEG_CHUNK,), jnp.int32),
        pltpu.VMEM((N_HIST,), jnp.float32),
        pltpu.VMEM((640,), jnp.float32),
        pltpu.VMEM((640,), jnp.float32),
        pltpu.VMEM_SHARED((NS, N_HIST), jnp.float32),
        pltpu.SemaphoreType.DMA((2,)),
    ],
    compiler_params=pltpu.CompilerParams(needs_layout_passes=False),
)


NBUF = 4


def _agg_body(h, srcp, dstp, out, sidx, didx, msg, zbuf, acc_sh,
              sisem, disem, gsem, ssem, zsem):
    c = lax.axis_index("c")
    s = lax.axis_index("s")

    zeros16 = jnp.zeros((16,), jnp.float32)
    for r in range(8):
        for l in range(D // 16):
            zbuf[r, pl.ds(l * 16, 16)] = zeros16
    zb = pl.multiple_of(s * ROWS_T, 8)
    for r in range(ROWS_T // 8):
        pltpu.async_copy(zbuf, acc_sh.at[pl.ds(zb + r * 8, 8)], zsem)
    for r in range(ROWS_T // 8):
        pltpu.make_async_copy(zbuf, acc_sh.at[pl.ds(zb, 8)], zsem).wait()
    plsc.subcore_barrier()

    t_len = T_C0 + c * (T_C1 - T_C0)
    eb = c * E_C0 + s * t_len
    n_chunks = t_len // AGG_CHUNK

    def _off(k):
        return pl.multiple_of(eb + k * AGG_CHUNK, 8)

    def _start_sidx(k, b):
        pltpu.async_copy(srcp.at[pl.ds(_off(k), AGG_CHUNK)], sidx.at[b], sisem.at[b])

    def _wait_sidx(b):
        pltpu.make_async_copy(srcp.at[pl.ds(0, AGG_CHUNK)], sidx.at[b], sisem.at[b]).wait()

    def _start_didx(k, b):
        pltpu.async_copy(dstp.at[pl.ds(_off(k), AGG_CHUNK)], didx.at[b], disem.at[b])

    def _wait_didx(b):
        pltpu.make_async_copy(dstp.at[pl.ds(0, AGG_CHUNK)], didx.at[b], disem.at[b]).wait()

    def _start_gather(b):
        pltpu.async_copy(h.at[sidx.at[b]], msg.at[b], gsem.at[b])

    def _wait_gather(b):
        pltpu.make_async_copy(h.at[sidx.at[b]], msg.at[b], gsem.at[b]).wait()

    def _start_scatter(b):
        pltpu.async_copy(msg.at[b], acc_sh.at[didx.at[b]], ssem.at[b], add=True)

    def _wait_scatter(b):
        pltpu.make_async_copy(msg.at[b], acc_sh.at[didx.at[b]], ssem.at[b]).wait()

    for b in range(NBUF):
        _start_sidx(b, b)
    _start_didx(0, 0)
    _start_didx(1, 1)
    _wait_sidx(0)
    _start_gather(0)
    _wait_sidx(1)
    _start_gather(1)

    def _iter(g, carry):
        for b in range(NBUF):
            k = NBUF * g + b
            b2 = (b + 2) % NBUF

            _wait_gather(b)
            _wait_didx(b)
            _start_scatter(b)

            @pl.when(k + 4 < n_chunks)
            def _():
                _start_sidx(k + 4, b)

            @pl.when(k >= 2)
            def _():
                _wait_scatter(b2)

            @pl.when(k + 2 < n_chunks)
            def _():
                _wait_sidx(b2)
                _start_gather(b2)
                _start_didx(k + 2, b2)

        return carry

    lax.fori_loop(0, n_chunks // NBUF, _iter, 0)
    _wait_scatter(NBUF - 2)
    _wait_scatter(NBUF - 1)
    plsc.subcore_barrier()

    ob = pl.multiple_of(s * ROWS_T, 8)
    pltpu.sync_copy(acc_sh.at[pl.ds(ob, ROWS_T)], out.at[c, pl.ds(ob, ROWS_T)])


_agg_call = pl.kernel(
    _agg_body,
    out_type=jax.ShapeDtypeStruct((2, N_ACC, D), jnp.float32),
    mesh=plsc.VectorSubcoreMesh(core_axis_name="c", subcore_axis_name="s"),
    scratch_types=[
        pltpu.VMEM((NBUF, AGG_CHUNK), jnp.int32),
        pltpu.VMEM((NBUF, AGG_CHUNK), jnp.int32),
        pltpu.VMEM((NBUF, AGG_CHUNK, D), jnp.float32),
        pltpu.VMEM((8, D), jnp.float32),
        pltpu.VMEM_SHARED((N_ACC, D), jnp.float32),
        pltpu.SemaphoreType.DMA((NBUF,)),
        pltpu.SemaphoreType.DMA((NBUF,)),
        pltpu.SemaphoreType.DMA((NBUF,)),
        pltpu.SemaphoreType.DMA((NBUF,)),
        pltpu.SemaphoreType.DMA,
    ],
    compiler_params=pltpu.CompilerParams(needs_layout_passes=False),
)


def _scale_body(x_ref, deg_ref, o_ref):
    norm = lax.rsqrt(jnp.maximum(deg_ref[...], 1.0))
    o_ref[...] = x_ref[...] * norm


def _mid_body(parts_ref, dd_ref, ds_ref, w_ref, o_ref):
    agg = parts_ref[0, :N_NODES, :] + parts_ref[1, :N_NODES, :]
    nd = lax.rsqrt(jnp.maximum(dd_ref[...], 1.0))
    ns = lax.rsqrt(jnp.maximum(ds_ref[...], 1.0))
    h = jnp.dot(agg * nd, w_ref[...], preferred_element_type=jnp.float32)
    o_ref[...] = jnp.maximum(h, 0.0) * ns


def _out_body(parts_ref, dd_ref, w_ref, o_ref):
    agg = parts_ref[0, :N_NODES, :] + parts_ref[1, :N_NODES, :]
    nd = lax.rsqrt(jnp.maximum(dd_ref[...], 1.0))
    o_ref[...] = jnp.dot(agg * nd, w_ref[...], preferred_element_type=jnp.float32)


_scale_call = pl.pallas_call(
    _scale_body,
    out_shape=jax.ShapeDtypeStruct((N_NODES, D), jnp.float32),
)

_mid_call = pl.pallas_call(
    _mid_body,
    out_shape=jax.ShapeDtypeStruct((N_NODES, D), jnp.float32),
)

_out_call = pl.pallas_call(
    _out_body,
    out_shape=jax.ShapeDtypeStruct((N_NODES, D), jnp.float32),
)


def kernel(x, edge_index, W1, W2):
    ei = edge_index.astype(jnp.int32)
    n_pad = E_PAD - N_EDGES
    srcp = jnp.concatenate([ei[0], jnp.zeros((n_pad,), jnp.int32)])
    dstp = jnp.concatenate([ei[1], jnp.full((n_pad,), N_NODES, jnp.int32)])
    edges_flat = jnp.concatenate([ei[0], ei[1]])
    degs = _deg_call(edges_flat)
    deg_src = degs[:N_NODES].reshape(N_NODES, 1)
    deg_dst = degs[N_HIST:N_HIST + N_NODES].reshape(N_NODES, 1)

    h0 = _scale_call(x, deg_src)
    parts1 = _agg_call(h0, srcp, dstp)
    h1 = _mid_call(parts1, deg_dst, deg_src, W1)
    parts2 = _agg_call(h1, srcp, dstp)
    return _out_call(parts2, deg_dst, W2)


# speedup vs baseline: 1.3601x; 1.2410x over previous
import jax
import jax.numpy as jnp
import numpy as np
from jax import lax
from jax.experimental import pallas as pl
from jax.experimental.pallas import tpu as pltpu
from jax.experimental.pallas import tpu_sc as plsc

N_NODES = 10000
N_EDGES = 320000
D = 128

NC = 2
NS = 16
NW = NC * NS

E_TILE_DEG = N_EDGES // NS
DEG_CHUNK = 2000
N_HIST = NS * 640

E_PAD = 327680
E_C0 = 245760
T_C0 = E_C0 // NS
T_C1 = (E_PAD - E_C0) // NS
AGG_CHUNK = 64
_ar = np.arange(D)
_base = (_ar // 32) * 32
_off = _ar - _base
_g = _base + np.where(_off < 16, 2 * _off, 2 * (_off - 16) + 1)
_P_NP = np.zeros((D, D), np.float32)
_P_NP[_ar, _g] = 1.0

N_ACC = NS * 632
ROWS_T = N_ACC // NS


def _deg_body(edges, out, ibuf0, ibuf1, hist, acc, tmp, shist, isem):
    ibufs = (ibuf0, ibuf1)
    c = lax.axis_index("c")
    s = lax.axis_index("s")
    zeros16 = jnp.zeros((16,), jnp.float32)
    ones16 = jnp.ones((16,), jnp.float32)

    def _zero(i, carry):
        hist[pl.ds(i * 16, 16)] = zeros16
        return carry

    lax.fori_loop(0, N_HIST // 16, _zero, 0)

    base = c * N_EDGES + s * E_TILE_DEG
    n_chunks = E_TILE_DEG // DEG_CHUNK

    def _start_idx(k, b):
        off = pl.multiple_of(base + k * DEG_CHUNK, 8)
        pltpu.async_copy(edges.at[pl.ds(off, DEG_CHUNK)], ibufs[b], isem.at[b])

    def _wait_idx(b):
        pltpu.make_async_copy(
            edges.at[pl.ds(0, DEG_CHUNK)], ibufs[b], isem.at[b]
        ).wait()

    _start_idx(0, 0)
    _start_idx(1, 1)

    def _chunk(g, carry):
        for b in (0, 1):
            k = 2 * g + b
            _wait_idx(b)

            ib = ibufs[b]

            def _scan(j, c2):
                idx = ib[pl.ds(j * 16, 16)]
                plsc.addupdate_scatter(hist, [idx], ones16)
                return c2

            lax.fori_loop(0, DEG_CHUNK // 16, _scan, 0)

            @pl.when(k + 2 < n_chunks)
            def _():
                _start_idx(k + 2, b)

        return carry

    lax.fori_loop(0, n_chunks // 2, _chunk, 0)

    pltpu.sync_copy(hist, shist.at[s])
    plsc.subcore_barrier()

    rb = pl.multiple_of(s * 640, 8)
    pltpu.sync_copy(shist.at[0, pl.ds(rb, 640)], acc)

    def _reduce(h, carry):
        pltpu.sync_copy(shist.at[h, pl.ds(rb, 640)], tmp)

        def _addv(i, c2):
            o = i * 16
            acc[pl.ds(o, 16)] = acc[pl.ds(o, 16)] + tmp[pl.ds(o, 16)]
            return c2

        lax.fori_loop(0, 640 // 16, _addv, 0)
        return carry

    lax.fori_loop(1, NS, _reduce, 0)
    ob = pl.multiple_of(c * N_HIST + s * 640, 8)
    pltpu.sync_copy(acc, out.at[pl.ds(ob, 640)])


_deg_call = pl.kernel(
    _deg_body,
    out_type=jax.ShapeDtypeStruct((2 * N_HIST,), jnp.float32),
    mesh=plsc.VectorSubcoreMesh(core_axis_name="c", subcore_axis_name="s"),
    scratch_types=[
        pltpu.VMEM((DEG_CHUNK,), jnp.int32),
        pltpu.VMEM((DEG_CHUNK,), jnp.int32),
        pltpu.VMEM((N_HIST,), jnp.float32),
        pltpu.VMEM((640,), jnp.float32),
        pltpu.VMEM((640,), jnp.float32),
        pltpu.VMEM_SHARED((NS, N_HIST), jnp.float32),
        pltpu.SemaphoreType.DMA((2,)),
    ],
    compiler_params=pltpu.CompilerParams(needs_layout_passes=False),
)


NBUF = 4


def _agg_body(hbf, srcp, dstp, out, sidx, didx, mbf, mf32, zbuf, acc_sh,
              sisem, disem, gsem, ssem, zsem):
    c = lax.axis_index("c")
    s = lax.axis_index("s")

    zeros16 = jnp.zeros((16,), jnp.float32)
    for r in range(8):
        for l in range(D // 16):
            zbuf[r, pl.ds(l * 16, 16)] = zeros16
    zb = pl.multiple_of(s * ROWS_T, 8)
    for r in range(ROWS_T // 8):
        pltpu.async_copy(zbuf, acc_sh.at[pl.ds(zb + r * 8, 8)], zsem)
    for r in range(ROWS_T // 8):
        pltpu.make_async_copy(zbuf, acc_sh.at[pl.ds(zb, 8)], zsem).wait()
    plsc.subcore_barrier()

    t_len = T_C0 + c * (T_C1 - T_C0)
    eb = c * E_C0 + s * t_len
    n_chunks = t_len // AGG_CHUNK

    def _off(k):
        return pl.multiple_of(eb + k * AGG_CHUNK, 8)

    def _start_sidx(k, b):
        pltpu.async_copy(srcp.at[pl.ds(_off(k), AGG_CHUNK)], sidx.at[b], sisem.at[b])

    def _wait_sidx(b):
        pltpu.make_async_copy(srcp.at[pl.ds(0, AGG_CHUNK)], sidx.at[b], sisem.at[b]).wait()

    def _start_didx(k, b):
        pltpu.async_copy(dstp.at[pl.ds(_off(k), AGG_CHUNK)], didx.at[b], disem.at[b])

    def _wait_didx(b):
        pltpu.make_async_copy(dstp.at[pl.ds(0, AGG_CHUNK)], didx.at[b], disem.at[b]).wait()

    def _start_gather(b):
        pltpu.async_copy(hbf.at[sidx.at[b]], mbf.at[b], gsem.at[b])

    def _wait_gather(b):
        pltpu.make_async_copy(hbf.at[sidx.at[b]], mbf.at[b], gsem.at[b]).wait()

    def _convert(b, m):
        def _row(r, carry):
            for cb in range(D // 32):
                ab = mbf[b, r, pl.ds(cb * 32, 32)]
                lo, hi = plsc.unpack(ab, format=plsc.PackFormat.INTERLEAVED)
                mf32[m, r, pl.ds(cb * 32, 16)] = lo
                mf32[m, r, pl.ds(cb * 32 + 16, 16)] = hi
            return carry

        lax.fori_loop(0, AGG_CHUNK, _row, 0)

    def _start_scatter(m, d):
        pltpu.async_copy(mf32.at[m], acc_sh.at[didx.at[d]], ssem.at[d], add=True)

    def _wait_scatter(m, d):
        pltpu.make_async_copy(mf32.at[m], acc_sh.at[didx.at[d]], ssem.at[d]).wait()

    for b in range(NBUF):
        _start_sidx(b, b)
    _start_didx(0, 0)
    _start_didx(1, 1)
    _wait_sidx(0)
    _start_gather(0)
    _wait_sidx(1)
    _start_gather(1)

    def _iter(g, carry):
        for b in range(NBUF):
            k = NBUF * g + b
            m = b % 2
            b2 = (b + 2) % NBUF

            _wait_gather(b)

            @pl.when(k >= 2)
            def _():
                _wait_scatter(m, b2)

            _convert(b, m)
            _wait_didx(b)
            _start_scatter(m, b)

            @pl.when(k + 4 < n_chunks)
            def _():
                _start_sidx(k + 4, b)

            @pl.when(k + 2 < n_chunks)
            def _():
                _wait_sidx(b2)
                _start_gather(b2)
                _start_didx(k + 2, b2)

        return carry

    lax.fori_loop(0, n_chunks // NBUF, _iter, 0)
    _wait_scatter(0, NBUF - 2)
    _wait_scatter(1, NBUF - 1)
    plsc.subcore_barrier()

    ob = pl.multiple_of(s * ROWS_T, 8)
    pltpu.sync_copy(acc_sh.at[pl.ds(ob, ROWS_T)], out.at[c, pl.ds(ob, ROWS_T)])


_agg_call = pl.kernel(
    _agg_body,
    out_type=jax.ShapeDtypeStruct((2, N_ACC, D), jnp.float32),
    mesh=plsc.VectorSubcoreMesh(core_axis_name="c", subcore_axis_name="s"),
    scratch_types=[
        pltpu.VMEM((NBUF, AGG_CHUNK), jnp.int32),
        pltpu.VMEM((NBUF, AGG_CHUNK), jnp.int32),
        pltpu.VMEM((NBUF, AGG_CHUNK, D), jnp.bfloat16),
        pltpu.VMEM((2, AGG_CHUNK, D), jnp.float32),
        pltpu.VMEM((8, D), jnp.float32),
        pltpu.VMEM_SHARED((N_ACC, D), jnp.float32),
        pltpu.SemaphoreType.DMA((NBUF,)),
        pltpu.SemaphoreType.DMA((NBUF,)),
        pltpu.SemaphoreType.DMA((NBUF,)),
        pltpu.SemaphoreType.DMA((NBUF,)),
        pltpu.SemaphoreType.DMA,
    ],
    compiler_params=pltpu.CompilerParams(
        needs_layout_passes=False, use_tc_tiling_on_sc=False),
)


def _scale_body(x_ref, deg_ref, p_ref, o_ref):
    norm = lax.rsqrt(jnp.maximum(deg_ref[...], 1.0))
    v = jnp.dot(x_ref[...] * norm, p_ref[...], preferred_element_type=jnp.float32)
    o_ref[...] = v.astype(jnp.bfloat16)


def _mid_body(parts_ref, dd_ref, ds_ref, w_ref, p_ref, o_ref):
    agg = parts_ref[0, :N_NODES, :] + parts_ref[1, :N_NODES, :]
    nd = lax.rsqrt(jnp.maximum(dd_ref[...], 1.0))
    ns = lax.rsqrt(jnp.maximum(ds_ref[...], 1.0))
    h = jnp.dot(agg * nd, w_ref[...], preferred_element_type=jnp.float32)
    v = jnp.dot(jnp.maximum(h, 0.0) * ns, p_ref[...],
                preferred_element_type=jnp.float32)
    o_ref[...] = v.astype(jnp.bfloat16)


def _out_body(parts_ref, dd_ref, w_ref, o_ref):
    agg = parts_ref[0, :N_NODES, :] + parts_ref[1, :N_NODES, :]
    nd = lax.rsqrt(jnp.maximum(dd_ref[...], 1.0))
    o_ref[...] = jnp.dot(agg * nd, w_ref[...], preferred_element_type=jnp.float32)


_scale_call = pl.pallas_call(
    _scale_body,
    out_shape=jax.ShapeDtypeStruct((N_NODES, D), jnp.bfloat16),
)

_mid_call = pl.pallas_call(
    _mid_body,
    out_shape=jax.ShapeDtypeStruct((N_NODES, D), jnp.bfloat16),
)

_out_call = pl.pallas_call(
    _out_body,
    out_shape=jax.ShapeDtypeStruct((N_NODES, D), jnp.float32),
)


def kernel(x, edge_index, W1, W2):
    ei = edge_index.astype(jnp.int32)
    n_pad = E_PAD - N_EDGES
    srcp = jnp.concatenate([ei[0], jnp.zeros((n_pad,), jnp.int32)])
    dstp = jnp.concatenate([ei[1], jnp.full((n_pad,), N_NODES, jnp.int32)])
    edges_flat = jnp.concatenate([ei[0], ei[1]])
    degs = _deg_call(edges_flat)
    deg_src = degs[:N_NODES].reshape(N_NODES, 1)
    deg_dst = degs[N_HIST:N_HIST + N_NODES].reshape(N_NODES, 1)

    P = jnp.asarray(_P_NP)
    h0bf = _scale_call(x, deg_src, P)
    parts1 = _agg_call(h0bf, srcp, dstp)
    h1bf = _mid_call(parts1, deg_dst, deg_src, W1, P)
    parts2 = _agg_call(h1bf, srcp, dstp)
    return _out_call(parts2, deg_dst, W2)


# speedup vs baseline: 1.5263x; 1.1222x over previous
import jax
import jax.numpy as jnp
import numpy as np
from jax import lax
from jax.experimental import pallas as pl
from jax.experimental.pallas import tpu as pltpu
from jax.experimental.pallas import tpu_sc as plsc

N_NODES = 10000
N_EDGES = 320000
D = 128

NC = 2
NS = 16
NW = NC * NS

E_TILE_DEG = N_EDGES // NS
DEG_CHUNK = 2000
N_HIST = NS * 640

E_PAD = 327680
E_C0 = 212992
T_C0 = E_C0 // NS
T_C1 = (E_PAD - E_C0) // NS
AGG_CHUNK = 64
_ar = np.arange(D)
_base = (_ar // 32) * 32
_off = _ar - _base
_g = _base + np.where(_off < 16, 2 * _off, 2 * (_off - 16) + 1)
_P_NP = np.zeros((D, D), np.float32)
_P_NP[_ar, _g] = 1.0

N_ACC = NS * 632
ROWS_T = N_ACC // NS


def _deg_body(edges, out, ibuf0, ibuf1, hist, acc, tmp, shist, isem):
    ibufs = (ibuf0, ibuf1)
    c = lax.axis_index("c")
    s = lax.axis_index("s")
    zeros16 = jnp.zeros((16,), jnp.float32)
    ones16 = jnp.ones((16,), jnp.float32)

    def _zero(i, carry):
        hist[pl.ds(i * 16, 16)] = zeros16
        return carry

    lax.fori_loop(0, N_HIST // 16, _zero, 0)

    base = c * N_EDGES + s * E_TILE_DEG
    n_chunks = E_TILE_DEG // DEG_CHUNK

    def _start_idx(k, b):
        off = pl.multiple_of(base + k * DEG_CHUNK, 8)
        pltpu.async_copy(edges.at[pl.ds(off, DEG_CHUNK)], ibufs[b], isem.at[b])

    def _wait_idx(b):
        pltpu.make_async_copy(
            edges.at[pl.ds(0, DEG_CHUNK)], ibufs[b], isem.at[b]
        ).wait()

    _start_idx(0, 0)
    _start_idx(1, 1)

    def _chunk(g, carry):
        for b in (0, 1):
            k = 2 * g + b
            _wait_idx(b)

            ib = ibufs[b]

            def _scan(j, c2):
                idx = ib[pl.ds(j * 16, 16)]
                plsc.addupdate_scatter(hist, [idx], ones16)
                return c2

            lax.fori_loop(0, DEG_CHUNK // 16, _scan, 0)

            @pl.when(k + 2 < n_chunks)
            def _():
                _start_idx(k + 2, b)

        return carry

    lax.fori_loop(0, n_chunks // 2, _chunk, 0)

    pltpu.sync_copy(hist, shist.at[s])
    plsc.subcore_barrier()

    rb = pl.multiple_of(s * 640, 8)
    pltpu.sync_copy(shist.at[0, pl.ds(rb, 640)], acc)

    def _reduce(h, carry):
        pltpu.sync_copy(shist.at[h, pl.ds(rb, 640)], tmp)

        def _addv(i, c2):
            o = i * 16
            acc[pl.ds(o, 16)] = acc[pl.ds(o, 16)] + tmp[pl.ds(o, 16)]
            return c2

        lax.fori_loop(0, 640 // 16, _addv, 0)
        return carry

    lax.fori_loop(1, NS, _reduce, 0)
    ob = pl.multiple_of(c * N_HIST + s * 640, 8)
    pltpu.sync_copy(acc, out.at[pl.ds(ob, 640)])


_deg_call = pl.kernel(
    _deg_body,
    out_type=jax.ShapeDtypeStruct((2 * N_HIST,), jnp.float32),
    mesh=plsc.VectorSubcoreMesh(core_axis_name="c", subcore_axis_name="s"),
    scratch_types=[
        pltpu.VMEM((DEG_CHUNK,), jnp.int32),
        pltpu.VMEM((DEG_CHUNK,), jnp.int32),
        pltpu.VMEM((N_HIST,), jnp.float32),
        pltpu.VMEM((640,), jnp.float32),
        pltpu.VMEM((640,), jnp.float32),
        pltpu.VMEM_SHARED((NS, N_HIST), jnp.float32),
        pltpu.SemaphoreType.DMA((2,)),
    ],
    compiler_params=pltpu.CompilerParams(needs_layout_passes=False),
)


NBUF = 4


def _agg_body(hbf, srcp, dstp, out, sidx, didx, mbf, mf32, zbuf, acc_sh,
              sisem, disem, gsem, ssem, zsem):
    c = lax.axis_index("c")
    s = lax.axis_index("s")

    zeros16 = jnp.zeros((16,), jnp.float32)
    for r in range(8):
        for l in range(D // 16):
            zbuf[r, pl.ds(l * 16, 16)] = zeros16
    zb = pl.multiple_of(s * ROWS_T, 8)
    for r in range(ROWS_T // 8):
        pltpu.async_copy(zbuf, acc_sh.at[pl.ds(zb + r * 8, 8)], zsem)
    for r in range(ROWS_T // 8):
        pltpu.make_async_copy(zbuf, acc_sh.at[pl.ds(zb, 8)], zsem).wait()
    plsc.subcore_barrier()

    t_len = T_C0 + c * (T_C1 - T_C0)
    eb = c * E_C0 + s * t_len
    n_chunks = t_len // AGG_CHUNK

    def _off(k):
        return pl.multiple_of(eb + k * AGG_CHUNK, 8)

    def _start_sidx(k, b):
        pltpu.async_copy(srcp.at[pl.ds(_off(k), AGG_CHUNK)], sidx.at[b], sisem.at[b])

    def _wait_sidx(b):
        pltpu.make_async_copy(srcp.at[pl.ds(0, AGG_CHUNK)], sidx.at[b], sisem.at[b]).wait()

    def _start_didx(k, b):
        pltpu.async_copy(dstp.at[pl.ds(_off(k), AGG_CHUNK)], didx.at[b], disem.at[b])

    def _wait_didx(b):
        pltpu.make_async_copy(dstp.at[pl.ds(0, AGG_CHUNK)], didx.at[b], disem.at[b]).wait()

    def _start_gather(b):
        pltpu.async_copy(hbf.at[sidx.at[b]], mbf.at[b], gsem.at[b])

    def _wait_gather(b):
        pltpu.make_async_copy(hbf.at[sidx.at[b]], mbf.at[b], gsem.at[b]).wait()

    def _convert(b, m):
        def _row(r, carry):
            for cb in range(D // 32):
                ab = mbf[b, r, pl.ds(cb * 32, 32)]
                lo, hi = plsc.unpack(ab, format=plsc.PackFormat.INTERLEAVED)
                mf32[m, r, pl.ds(cb * 32, 16)] = lo
                mf32[m, r, pl.ds(cb * 32 + 16, 16)] = hi
            return carry

        lax.fori_loop(0, AGG_CHUNK, _row, 0)

    def _start_scatter(m, d):
        pltpu.async_copy(mf32.at[m], acc_sh.at[didx.at[d]], ssem.at[d], add=True)

    def _wait_scatter(m, d):
        pltpu.make_async_copy(mf32.at[m], acc_sh.at[didx.at[d]], ssem.at[d]).wait()

    for b in range(NBUF):
        _start_sidx(b, b)
    _start_didx(0, 0)
    _start_didx(1, 1)
    _wait_sidx(0)
    _start_gather(0)
    _wait_sidx(1)
    _start_gather(1)

    def _iter(g, carry):
        for b in range(NBUF):
            k = NBUF * g + b
            m = b % 2
            b2 = (b + 2) % NBUF

            _wait_gather(b)

            @pl.when(k >= 2)
            def _():
                _wait_scatter(m, b2)

            _convert(b, m)
            _wait_didx(b)
            _start_scatter(m, b)

            @pl.when(k + 4 < n_chunks)
            def _():
                _start_sidx(k + 4, b)

            @pl.when(k + 2 < n_chunks)
            def _():
                _wait_sidx(b2)
                _start_gather(b2)
                _start_didx(k + 2, b2)

        return carry

    lax.fori_loop(0, n_chunks // NBUF, _iter, 0)
    _wait_scatter(0, NBUF - 2)
    _wait_scatter(1, NBUF - 1)
    plsc.subcore_barrier()

    ob = pl.multiple_of(s * ROWS_T, 8)
    pltpu.sync_copy(acc_sh.at[pl.ds(ob, ROWS_T)], out.at[c, pl.ds(ob, ROWS_T)])


_agg_call = pl.kernel(
    _agg_body,
    out_type=jax.ShapeDtypeStruct((2, N_ACC, D), jnp.float32),
    mesh=plsc.VectorSubcoreMesh(core_axis_name="c", subcore_axis_name="s"),
    scratch_types=[
        pltpu.VMEM((NBUF, AGG_CHUNK), jnp.int32),
        pltpu.VMEM((NBUF, AGG_CHUNK), jnp.int32),
        pltpu.VMEM((NBUF, AGG_CHUNK, D), jnp.bfloat16),
        pltpu.VMEM((2, AGG_CHUNK, D), jnp.float32),
        pltpu.VMEM((8, D), jnp.float32),
        pltpu.VMEM_SHARED((N_ACC, D), jnp.float32),
        pltpu.SemaphoreType.DMA((NBUF,)),
        pltpu.SemaphoreType.DMA((NBUF,)),
        pltpu.SemaphoreType.DMA((NBUF,)),
        pltpu.SemaphoreType.DMA((NBUF,)),
        pltpu.SemaphoreType.DMA,
    ],
    compiler_params=pltpu.CompilerParams(
        needs_layout_passes=False, use_tc_tiling_on_sc=False),
)


def _scale_body(x_ref, deg_ref, p_ref, o_ref):
    norm = lax.rsqrt(jnp.maximum(deg_ref[...], 1.0))
    v = jnp.dot(x_ref[...] * norm, p_ref[...], preferred_element_type=jnp.float32)
    o_ref[...] = v.astype(jnp.bfloat16)


def _mid_body(parts_ref, dd_ref, ds_ref, w_ref, p_ref, o_ref):
    agg = parts_ref[0, :N_NODES, :] + parts_ref[1, :N_NODES, :]
    nd = lax.rsqrt(jnp.maximum(dd_ref[...], 1.0))
    ns = lax.rsqrt(jnp.maximum(ds_ref[...], 1.0))
    h = jnp.dot(agg * nd, w_ref[...], preferred_element_type=jnp.float32)
    v = jnp.dot(jnp.maximum(h, 0.0) * ns, p_ref[...],
                preferred_element_type=jnp.float32)
    o_ref[...] = v.astype(jnp.bfloat16)


def _out_body(parts_ref, dd_ref, w_ref, o_ref):
    agg = parts_ref[0, :N_NODES, :] + parts_ref[1, :N_NODES, :]
    nd = lax.rsqrt(jnp.maximum(dd_ref[...], 1.0))
    o_ref[...] = jnp.dot(agg * nd, w_ref[...], preferred_element_type=jnp.float32)


_scale_call = pl.pallas_call(
    _scale_body,
    out_shape=jax.ShapeDtypeStruct((N_NODES, D), jnp.bfloat16),
)

_mid_call = pl.pallas_call(
    _mid_body,
    out_shape=jax.ShapeDtypeStruct((N_NODES, D), jnp.bfloat16),
)

_out_call = pl.pallas_call(
    _out_body,
    out_shape=jax.ShapeDtypeStruct((N_NODES, D), jnp.float32),
)


def kernel(x, edge_index, W1, W2):
    ei = edge_index.astype(jnp.int32)
    n_pad = E_PAD - N_EDGES
    srcp = jnp.concatenate([ei[0], jnp.zeros((n_pad,), jnp.int32)])
    dstp = jnp.concatenate([ei[1], jnp.full((n_pad,), N_NODES, jnp.int32)])
    edges_flat = jnp.concatenate([ei[0], ei[1]])
    degs = _deg_call(edges_flat)
    deg_src = degs[:N_NODES].reshape(N_NODES, 1)
    deg_dst = degs[N_HIST:N_HIST + N_NODES].reshape(N_NODES, 1)

    P = jnp.asarray(_P_NP)
    h0bf = _scale_call(x, deg_src, P)
    parts1 = _agg_call(h0bf, srcp, dstp)
    h1bf = _mid_call(parts1, deg_dst, deg_src, W1, P)
    parts2 = _agg_call(h1bf, srcp, dstp)
    return _out_call(parts2, deg_dst, W2)


# speedup vs baseline: 1.6032x; 1.0504x over previous
import jax
import jax.numpy as jnp
import numpy as np
from jax import lax
from jax.experimental import pallas as pl
from jax.experimental.pallas import tpu as pltpu
from jax.experimental.pallas import tpu_sc as plsc

N_NODES = 10000
N_EDGES = 320000
D = 128

NC = 2
NS = 16
NW = NC * NS

E_TILE_DEG = N_EDGES // NS
DEG_CHUNK = 2000
N_HIST = NS * 640

E_PAD = 327680
E_C0 = 196608
T_C0 = E_C0 // NS
T_C1 = (E_PAD - E_C0) // NS
AGG_CHUNK = 64
_ar = np.arange(D)
_base = (_ar // 32) * 32
_off = _ar - _base
_g = _base + np.where(_off < 16, 2 * _off, 2 * (_off - 16) + 1)
_P_NP = np.zeros((D, D), np.float32)
_P_NP[_ar, _g] = 1.0

N_ACC = NS * 632
ROWS_T = N_ACC // NS


def _deg_body(edges, out, ibuf0, ibuf1, hist, acc, tmp, shist, isem):
    ibufs = (ibuf0, ibuf1)
    c = lax.axis_index("c")
    s = lax.axis_index("s")
    zeros16 = jnp.zeros((16,), jnp.float32)
    ones16 = jnp.ones((16,), jnp.float32)

    def _zero(i, carry):
        hist[pl.ds(i * 16, 16)] = zeros16
        return carry

    lax.fori_loop(0, N_HIST // 16, _zero, 0)

    base = c * N_EDGES + s * E_TILE_DEG
    n_chunks = E_TILE_DEG // DEG_CHUNK

    def _start_idx(k, b):
        off = pl.multiple_of(base + k * DEG_CHUNK, 8)
        pltpu.async_copy(edges.at[pl.ds(off, DEG_CHUNK)], ibufs[b], isem.at[b])

    def _wait_idx(b):
        pltpu.make_async_copy(
            edges.at[pl.ds(0, DEG_CHUNK)], ibufs[b], isem.at[b]
        ).wait()

    _start_idx(0, 0)
    _start_idx(1, 1)

    def _chunk(g, carry):
        for b in (0, 1):
            k = 2 * g + b
            _wait_idx(b)

            ib = ibufs[b]

            def _scan(j, c2):
                idx = ib[pl.ds(j * 16, 16)]
                plsc.addupdate_scatter(hist, [idx], ones16)
                return c2

            lax.fori_loop(0, DEG_CHUNK // 16, _scan, 0)

            @pl.when(k + 2 < n_chunks)
            def _():
                _start_idx(k + 2, b)

        return carry

    lax.fori_loop(0, n_chunks // 2, _chunk, 0)

    pltpu.sync_copy(hist, shist.at[s])
    plsc.subcore_barrier()

    rb = pl.multiple_of(s * 640, 8)
    pltpu.sync_copy(shist.at[0, pl.ds(rb, 640)], acc)

    def _reduce(h, carry):
        pltpu.sync_copy(shist.at[h, pl.ds(rb, 640)], tmp)

        def _addv(i, c2):
            o = i * 16
            acc[pl.ds(o, 16)] = acc[pl.ds(o, 16)] + tmp[pl.ds(o, 16)]
            return c2

        lax.fori_loop(0, 640 // 16, _addv, 0)
        return carry

    lax.fori_loop(1, NS, _reduce, 0)
    ob = pl.multiple_of(c * N_HIST + s * 640, 8)
    pltpu.sync_copy(acc, out.at[pl.ds(ob, 640)])


_deg_call = pl.kernel(
    _deg_body,
    out_type=jax.ShapeDtypeStruct((2 * N_HIST,), jnp.float32),
    mesh=plsc.VectorSubcoreMesh(core_axis_name="c", subcore_axis_name="s"),
    scratch_types=[
        pltpu.VMEM((DEG_CHUNK,), jnp.int32),
        pltpu.VMEM((DEG_CHUNK,), jnp.int32),
        pltpu.VMEM((N_HIST,), jnp.float32),
        pltpu.VMEM((640,), jnp.float32),
        pltpu.VMEM((640,), jnp.float32),
        pltpu.VMEM_SHARED((NS, N_HIST), jnp.float32),
        pltpu.SemaphoreType.DMA((2,)),
    ],
    compiler_params=pltpu.CompilerParams(needs_layout_passes=False),
)


NBUF = 4


def _agg_body(hbf, srcp, dstp, out, sidx, didx, mbf, mf32, zbuf, acc_sh,
              sisem, disem, gsem, ssem, zsem):
    c = lax.axis_index("c")
    s = lax.axis_index("s")

    zeros16 = jnp.zeros((16,), jnp.float32)
    for r in range(8):
        for l in range(D // 16):
            zbuf[r, pl.ds(l * 16, 16)] = zeros16
    zb = pl.multiple_of(s * ROWS_T, 8)
    for r in range(ROWS_T // 8):
        pltpu.async_copy(zbuf, acc_sh.at[pl.ds(zb + r * 8, 8)], zsem)
    for r in range(ROWS_T // 8):
        pltpu.make_async_copy(zbuf, acc_sh.at[pl.ds(zb, 8)], zsem).wait()
    plsc.subcore_barrier()

    t_len = T_C0 + c * (T_C1 - T_C0)
    eb = c * E_C0 + s * t_len
    n_chunks = t_len // AGG_CHUNK

    def _off(k):
        return pl.multiple_of(eb + k * AGG_CHUNK, 8)

    def _start_sidx(k, b):
        pltpu.async_copy(srcp.at[pl.ds(_off(k), AGG_CHUNK)], sidx.at[b], sisem.at[b])

    def _wait_sidx(b):
        pltpu.make_async_copy(srcp.at[pl.ds(0, AGG_CHUNK)], sidx.at[b], sisem.at[b]).wait()

    def _start_didx(k, b):
        pltpu.async_copy(dstp.at[pl.ds(_off(k), AGG_CHUNK)], didx.at[b], disem.at[b])

    def _wait_didx(b):
        pltpu.make_async_copy(dstp.at[pl.ds(0, AGG_CHUNK)], didx.at[b], disem.at[b]).wait()

    def _start_gather(b):
        pltpu.async_copy(hbf.at[sidx.at[b]], mbf.at[b], gsem.at[b])

    def _wait_gather(b):
        pltpu.make_async_copy(hbf.at[sidx.at[b]], mbf.at[b], gsem.at[b]).wait()

    def _convert(b, m):
        def _row(r, carry):
            for cb in range(D // 32):
                ab = mbf[b, r, pl.ds(cb * 32, 32)]
                lo, hi = plsc.unpack(ab, format=plsc.PackFormat.INTERLEAVED)
                mf32[m, r, pl.ds(cb * 32, 16)] = lo
                mf32[m, r, pl.ds(cb * 32 + 16, 16)] = hi
            return carry

        lax.fori_loop(0, AGG_CHUNK, _row, 0)

    def _start_scatter(m, d):
        pltpu.async_copy(mf32.at[m], acc_sh.at[didx.at[d]], ssem.at[d], add=True)

    def _wait_scatter(m, d):
        pltpu.make_async_copy(mf32.at[m], acc_sh.at[didx.at[d]], ssem.at[d]).wait()

    for b in range(NBUF):
        _start_sidx(b, b)
    _start_didx(0, 0)
    _start_didx(1, 1)
    _wait_sidx(0)
    _start_gather(0)
    _wait_sidx(1)
    _start_gather(1)

    def _iter(g, carry):
        for b in range(NBUF):
            k = NBUF * g + b
            m = b % 2
            b2 = (b + 2) % NBUF

            _wait_gather(b)

            @pl.when(k >= 2)
            def _():
                _wait_scatter(m, b2)

            _convert(b, m)
            _wait_didx(b)
            _start_scatter(m, b)

            @pl.when(k + 4 < n_chunks)
            def _():
                _start_sidx(k + 4, b)

            @pl.when(k + 2 < n_chunks)
            def _():
                _wait_sidx(b2)
                _start_gather(b2)
                _start_didx(k + 2, b2)

        return carry

    lax.fori_loop(0, n_chunks // NBUF, _iter, 0)
    _wait_scatter(0, NBUF - 2)
    _wait_scatter(1, NBUF - 1)
    plsc.subcore_barrier()

    ob = pl.multiple_of(s * ROWS_T, 8)
    pltpu.sync_copy(acc_sh.at[pl.ds(ob, ROWS_T)], out.at[c, pl.ds(ob, ROWS_T)])


_agg_call = pl.kernel(
    _agg_body,
    out_type=jax.ShapeDtypeStruct((2, N_ACC, D), jnp.float32),
    mesh=plsc.VectorSubcoreMesh(core_axis_name="c", subcore_axis_name="s"),
    scratch_types=[
        pltpu.VMEM((NBUF, AGG_CHUNK), jnp.int32),
        pltpu.VMEM((NBUF, AGG_CHUNK), jnp.int32),
        pltpu.VMEM((NBUF, AGG_CHUNK, D), jnp.bfloat16),
        pltpu.VMEM((2, AGG_CHUNK, D), jnp.float32),
        pltpu.VMEM((8, D), jnp.float32),
        pltpu.VMEM_SHARED((N_ACC, D), jnp.float32),
        pltpu.SemaphoreType.DMA((NBUF,)),
        pltpu.SemaphoreType.DMA((NBUF,)),
        pltpu.SemaphoreType.DMA((NBUF,)),
        pltpu.SemaphoreType.DMA((NBUF,)),
        pltpu.SemaphoreType.DMA,
    ],
    compiler_params=pltpu.CompilerParams(
        needs_layout_passes=False, use_tc_tiling_on_sc=False),
)


def _scale_body(x_ref, deg_ref, p_ref, o_ref):
    norm = lax.rsqrt(jnp.maximum(deg_ref[...], 1.0))
    v = jnp.dot(x_ref[...] * norm, p_ref[...], preferred_element_type=jnp.float32)
    o_ref[...] = v.astype(jnp.bfloat16)


def _mid_body(parts_ref, dd_ref, ds_ref, w_ref, p_ref, o_ref):
    agg = parts_ref[0, :N_NODES, :] + parts_ref[1, :N_NODES, :]
    nd = lax.rsqrt(jnp.maximum(dd_ref[...], 1.0))
    ns = lax.rsqrt(jnp.maximum(ds_ref[...], 1.0))
    h = jnp.dot(agg * nd, w_ref[...], preferred_element_type=jnp.float32)
    v = jnp.dot(jnp.maximum(h, 0.0) * ns, p_ref[...],
                preferred_element_type=jnp.float32)
    o_ref[...] = v.astype(jnp.bfloat16)


def _out_body(parts_ref, dd_ref, w_ref, o_ref):
    agg = parts_ref[0, :N_NODES, :] + parts_ref[1, :N_NODES, :]
    nd = lax.rsqrt(jnp.maximum(dd_ref[...], 1.0))
    o_ref[...] = jnp.dot(agg * nd, w_ref[...], preferred_element_type=jnp.float32)


_scale_call = pl.pallas_call(
    _scale_body,
    out_shape=jax.ShapeDtypeStruct((N_NODES, D), jnp.bfloat16),
)

_mid_call = pl.pallas_call(
    _mid_body,
    out_shape=jax.ShapeDtypeStruct((N_NODES, D), jnp.bfloat16),
)

_out_call = pl.pallas_call(
    _out_body,
    out_shape=jax.ShapeDtypeStruct((N_NODES, D), jnp.float32),
)


def kernel(x, edge_index, W1, W2):
    ei = edge_index.astype(jnp.int32)
    n_pad = E_PAD - N_EDGES
    srcp = jnp.concatenate([ei[0], jnp.zeros((n_pad,), jnp.int32)])
    dstp = jnp.concatenate([ei[1], jnp.full((n_pad,), N_NODES, jnp.int32)])
    edges_flat = jnp.concatenate([ei[0], ei[1]])
    degs = _deg_call(edges_flat)
    deg_src = degs[:N_NODES].reshape(N_NODES, 1)
    deg_dst = degs[N_HIST:N_HIST + N_NODES].reshape(N_NODES, 1)

    P = jnp.asarray(_P_NP)
    h0bf = _scale_call(x, deg_src, P)
    parts1 = _agg_call(h0bf, srcp, dstp)
    h1bf = _mid_call(parts1, deg_dst, deg_src, W1, P)
    parts2 = _agg_call(h1bf, srcp, dstp)
    return _out_call(parts2, deg_dst, W2)


# speedup vs baseline: 1.6196x; 1.0102x over previous
import jax
import jax.numpy as jnp
import numpy as np
from jax import lax
from jax.experimental import pallas as pl
from jax.experimental.pallas import tpu as pltpu
from jax.experimental.pallas import tpu_sc as plsc

N_NODES = 10000
N_EDGES = 320000
D = 128

NC = 2
NS = 16
NW = NC * NS

E_TILE_DEG = N_EDGES // NS
DEG_CHUNK = 2000
N_HIST = NS * 640

E_PAD = 327680
E_C0 = 180224
T_C0 = E_C0 // NS
T_C1 = (E_PAD - E_C0) // NS
AGG_CHUNK = 64
_ar = np.arange(D)
_base = (_ar // 32) * 32
_off = _ar - _base
_g = _base + np.where(_off < 16, 2 * _off, 2 * (_off - 16) + 1)
_P_NP = np.zeros((D, D), np.float32)
_P_NP[_ar, _g] = 1.0

N_ACC = NS * 632
ROWS_T = N_ACC // NS


def _deg_body(edges, out, ibuf0, ibuf1, hist, acc, tmp, shist, isem):
    ibufs = (ibuf0, ibuf1)
    c = lax.axis_index("c")
    s = lax.axis_index("s")
    zeros16 = jnp.zeros((16,), jnp.float32)
    ones16 = jnp.ones((16,), jnp.float32)

    def _zero(i, carry):
        hist[pl.ds(i * 16, 16)] = zeros16
        return carry

    lax.fori_loop(0, N_HIST // 16, _zero, 0)

    base = c * N_EDGES + s * E_TILE_DEG
    n_chunks = E_TILE_DEG // DEG_CHUNK

    def _start_idx(k, b):
        off = pl.multiple_of(base + k * DEG_CHUNK, 8)
        pltpu.async_copy(edges.at[pl.ds(off, DEG_CHUNK)], ibufs[b], isem.at[b])

    def _wait_idx(b):
        pltpu.make_async_copy(
            edges.at[pl.ds(0, DEG_CHUNK)], ibufs[b], isem.at[b]
        ).wait()

    _start_idx(0, 0)
    _start_idx(1, 1)

    def _chunk(g, carry):
        for b in (0, 1):
            k = 2 * g + b
            _wait_idx(b)

            ib = ibufs[b]

            def _scan(j, c2):
                idx = ib[pl.ds(j * 16, 16)]
                plsc.addupdate_scatter(hist, [idx], ones16)
                return c2

            lax.fori_loop(0, DEG_CHUNK // 16, _scan, 0)

            @pl.when(k + 2 < n_chunks)
            def _():
                _start_idx(k + 2, b)

        return carry

    lax.fori_loop(0, n_chunks // 2, _chunk, 0)

    pltpu.sync_copy(hist, shist.at[s])
    plsc.subcore_barrier()

    rb = pl.multiple_of(s * 640, 8)
    pltpu.sync_copy(shist.at[0, pl.ds(rb, 640)], acc)

    def _reduce(h, carry):
        pltpu.sync_copy(shist.at[h, pl.ds(rb, 640)], tmp)

        def _addv(i, c2):
            o = i * 16
            acc[pl.ds(o, 16)] = acc[pl.ds(o, 16)] + tmp[pl.ds(o, 16)]
            return c2

        lax.fori_loop(0, 640 // 16, _addv, 0)
        return carry

    lax.fori_loop(1, NS, _reduce, 0)
    ob = pl.multiple_of(c * N_HIST + s * 640, 8)
    pltpu.sync_copy(acc, out.at[pl.ds(ob, 640)])


_deg_call = pl.kernel(
    _deg_body,
    out_type=jax.ShapeDtypeStruct((2 * N_HIST,), jnp.float32),
    mesh=plsc.VectorSubcoreMesh(core_axis_name="c", subcore_axis_name="s"),
    scratch_types=[
        pltpu.VMEM((DEG_CHUNK,), jnp.int32),
        pltpu.VMEM((DEG_CHUNK,), jnp.int32),
        pltpu.VMEM((N_HIST,), jnp.float32),
        pltpu.VMEM((640,), jnp.float32),
        pltpu.VMEM((640,), jnp.float32),
        pltpu.VMEM_SHARED((NS, N_HIST), jnp.float32),
        pltpu.SemaphoreType.DMA((2,)),
    ],
    compiler_params=pltpu.CompilerParams(needs_layout_passes=False),
)


NBUF = 4


def _agg_body(hbf, srcp, dstp, out, sidx, didx, mbf, mf32, zbuf, acc_sh,
              sisem, disem, gsem, ssem, zsem):
    c = lax.axis_index("c")
    s = lax.axis_index("s")

    zeros16 = jnp.zeros((16,), jnp.float32)
    for r in range(8):
        for l in range(D // 16):
            zbuf[r, pl.ds(l * 16, 16)] = zeros16
    zb = pl.multiple_of(s * ROWS_T, 8)
    for r in range(ROWS_T // 8):
        pltpu.async_copy(zbuf, acc_sh.at[pl.ds(zb + r * 8, 8)], zsem)
    for r in range(ROWS_T // 8):
        pltpu.make_async_copy(zbuf, acc_sh.at[pl.ds(zb, 8)], zsem).wait()
    plsc.subcore_barrier()

    t_len = T_C0 + c * (T_C1 - T_C0)
    eb = c * E_C0 + s * t_len
    n_chunks = t_len // AGG_CHUNK

    def _off(k):
        return pl.multiple_of(eb + k * AGG_CHUNK, 8)

    def _start_sidx(k, b):
        pltpu.async_copy(srcp.at[pl.ds(_off(k), AGG_CHUNK)], sidx.at[b], sisem.at[b])

    def _wait_sidx(b):
        pltpu.make_async_copy(srcp.at[pl.ds(0, AGG_CHUNK)], sidx.at[b], sisem.at[b]).wait()

    def _start_didx(k, b):
        pltpu.async_copy(dstp.at[pl.ds(_off(k), AGG_CHUNK)], didx.at[b], disem.at[b])

    def _wait_didx(b):
        pltpu.make_async_copy(dstp.at[pl.ds(0, AGG_CHUNK)], didx.at[b], disem.at[b]).wait()

    def _start_gather(b):
        pltpu.async_copy(hbf.at[sidx.at[b]], mbf.at[b], gsem.at[b])

    def _wait_gather(b):
        pltpu.make_async_copy(hbf.at[sidx.at[b]], mbf.at[b], gsem.at[b]).wait()

    def _convert(b, m):
        def _row(r, carry):
            for cb in range(D // 32):
                ab = mbf[b, r, pl.ds(cb * 32, 32)]
                lo, hi = plsc.unpack(ab, format=plsc.PackFormat.INTERLEAVED)
                mf32[m, r, pl.ds(cb * 32, 16)] = lo
                mf32[m, r, pl.ds(cb * 32 + 16, 16)] = hi
            return carry

        lax.fori_loop(0, AGG_CHUNK, _row, 0)

    def _start_scatter(m, d):
        pltpu.async_copy(mf32.at[m], acc_sh.at[didx.at[d]], ssem.at[d], add=True)

    def _wait_scatter(m, d):
        pltpu.make_async_copy(mf32.at[m], acc_sh.at[didx.at[d]], ssem.at[d]).wait()

    for b in range(NBUF):
        _start_sidx(b, b)
    _start_didx(0, 0)
    _start_didx(1, 1)
    _wait_sidx(0)
    _start_gather(0)
    _wait_sidx(1)
    _start_gather(1)

    def _iter(g, carry):
        for b in range(NBUF):
            k = NBUF * g + b
            m = b % 2
            b2 = (b + 2) % NBUF

            _wait_gather(b)

            @pl.when(k >= 2)
            def _():
                _wait_scatter(m, b2)

            _convert(b, m)
            _wait_didx(b)
            _start_scatter(m, b)

            @pl.when(k + 4 < n_chunks)
            def _():
                _start_sidx(k + 4, b)

            @pl.when(k + 2 < n_chunks)
            def _():
                _wait_sidx(b2)
                _start_gather(b2)
                _start_didx(k + 2, b2)

        return carry

    lax.fori_loop(0, n_chunks // NBUF, _iter, 0)
    _wait_scatter(0, NBUF - 2)
    _wait_scatter(1, NBUF - 1)
    plsc.subcore_barrier()

    ob = pl.multiple_of(s * ROWS_T, 8)
    pltpu.sync_copy(acc_sh.at[pl.ds(ob, ROWS_T)], out.at[c, pl.ds(ob, ROWS_T)])


_agg_call = pl.kernel(
    _agg_body,
    out_type=jax.ShapeDtypeStruct((2, N_ACC, D), jnp.float32),
    mesh=plsc.VectorSubcoreMesh(core_axis_name="c", subcore_axis_name="s"),
    scratch_types=[
        pltpu.VMEM((NBUF, AGG_CHUNK), jnp.int32),
        pltpu.VMEM((NBUF, AGG_CHUNK), jnp.int32),
        pltpu.VMEM((NBUF, AGG_CHUNK, D), jnp.bfloat16),
        pltpu.VMEM((2, AGG_CHUNK, D), jnp.float32),
        pltpu.VMEM((8, D), jnp.float32),
        pltpu.VMEM_SHARED((N_ACC, D), jnp.float32),
        pltpu.SemaphoreType.DMA((NBUF,)),
        pltpu.SemaphoreType.DMA((NBUF,)),
        pltpu.SemaphoreType.DMA((NBUF,)),
        pltpu.SemaphoreType.DMA((NBUF,)),
        pltpu.SemaphoreType.DMA,
    ],
    compiler_params=pltpu.CompilerParams(
        needs_layout_passes=False, use_tc_tiling_on_sc=False),
)


def _scale_body(x_ref, deg_ref, p_ref, o_ref):
    norm = lax.rsqrt(jnp.maximum(deg_ref[...], 1.0))
    v = jnp.dot(x_ref[...] * norm, p_ref[...], preferred_element_type=jnp.float32)
    o_ref[...] = v.astype(jnp.bfloat16)


def _mid_body(parts_ref, dd_ref, ds_ref, w_ref, p_ref, o_ref):
    agg = parts_ref[0, :N_NODES, :] + parts_ref[1, :N_NODES, :]
    nd = lax.rsqrt(jnp.maximum(dd_ref[...], 1.0))
    ns = lax.rsqrt(jnp.maximum(ds_ref[...], 1.0))
    h = jnp.dot(agg * nd, w_ref[...], preferred_element_type=jnp.float32)
    v = jnp.dot(jnp.maximum(h, 0.0) * ns, p_ref[...],
                preferred_element_type=jnp.float32)
    o_ref[...] = v.astype(jnp.bfloat16)


def _out_body(parts_ref, dd_ref, w_ref, o_ref):
    agg = parts_ref[0, :N_NODES, :] + parts_ref[1, :N_NODES, :]
    nd = lax.rsqrt(jnp.maximum(dd_ref[...], 1.0))
    o_ref[...] = jnp.dot(agg * nd, w_ref[...], preferred_element_type=jnp.float32)


_scale_call = pl.pallas_call(
    _scale_body,
    out_shape=jax.ShapeDtypeStruct((N_NODES, D), jnp.bfloat16),
)

_mid_call = pl.pallas_call(
    _mid_body,
    out_shape=jax.ShapeDtypeStruct((N_NODES, D), jnp.bfloat16),
)

_out_call = pl.pallas_call(
    _out_body,
    out_shape=jax.ShapeDtypeStruct((N_NODES, D), jnp.float32),
)


def kernel(x, edge_index, W1, W2):
    ei = edge_index.astype(jnp.int32)
    n_pad = E_PAD - N_EDGES
    srcp = jnp.concatenate([ei[0], jnp.zeros((n_pad,), jnp.int32)])
    dstp = jnp.concatenate([ei[1], jnp.full((n_pad,), N_NODES, jnp.int32)])
    edges_flat = jnp.concatenate([ei[0], ei[1]])
    degs = _deg_call(edges_flat)
    deg_src = degs[:N_NODES].reshape(N_NODES, 1)
    deg_dst = degs[N_HIST:N_HIST + N_NODES].reshape(N_NODES, 1)

    P = jnp.asarray(_P_NP)
    h0bf = _scale_call(x, deg_src, P)
    parts1 = _agg_call(h0bf, srcp, dstp)
    h1bf = _mid_call(parts1, deg_dst, deg_src, W1, P)
    parts2 = _agg_call(h1bf, srcp, dstp)
    return _out_call(parts2, deg_dst, W2)
